# Initial kernel scaffold; baseline (speedup 1.0000x reference)
import jax
import jax.numpy as jnp
from jax.experimental import pallas as pl


def kernel(pos, atom_features, edge_index, solvent_index, params):
    raise NotImplementedError("write your pallas kernel here")


# speedup vs baseline: 2.2879x; 2.2879x over previous
import functools

import jax
import jax.numpy as jnp
import numpy as np
from jax import lax
from jax.experimental import pallas as pl
from jax.experimental.pallas import tpu as pltpu
from jax.experimental.pallas import tpu_sc as plsc

N = 50000
E = 800000
HID = 64
RADIUS = 0.6
FRACTION = 0.1
SCALING = 2.0
EPS = 1e-6

EP = 802816
NCH = EP // 128
GCH = NCH // 32
SCH = NCH // 16
NT = N + 1
HALF = 25000
TROWS = 1664
ZCH = TROWS // 128
SROWS = 16 * TROWS
OUTR = 2 * SROWS

_MESH = dict(core_axis_name="c", subcore_axis_name="s", num_cores=2,
             num_subcores=16)
_SC_PARAMS = pltpu.CompilerParams(use_tc_tiling_on_sc=False)


def _f0(a):
    return np.zeros(a.shape, dtype=jax.dtypes.float0)


def _sc_gather(table, idx2d, D):
    mesh = plsc.VectorSubcoreMesh(**_MESH)

    def body(table_hbm, idx_hbm, out_hbm, idx_v, rows_v, sem):
        wid = lax.axis_index("s") * 2 + lax.axis_index("c")

        def chunk(i, carry):
            g = wid * GCH + i
            pltpu.sync_copy(idx_hbm.at[g], idx_v)
            pltpu.async_copy(table_hbm.at[idx_v], rows_v, sem).wait()
            pltpu.sync_copy(rows_v, out_hbm.at[pl.ds(g * 128, 128)])
            return carry

        lax.fori_loop(0, GCH, chunk, 0)

    return pl.kernel(
        body,
        out_type=jax.ShapeDtypeStruct((EP, D), jnp.float32),
        mesh=mesh,
        scratch_types=[
            pltpu.VMEM((128,), jnp.int32),
            pltpu.VMEM((128, D), jnp.float32),
            pltpu.SemaphoreType.DMA,
        ],
        compiler_params=_SC_PARAMS,
    )(table, idx2d)


def _sc_scatter(vals, idx2d, D):
    mesh = plsc.VectorSubcoreMesh(**_MESH)

    def body(vals_hbm, idx_hbm, out_hbm, idx_v, lidx_v, vals_v, acc_sh, sem):
        c = lax.axis_index("c")
        s = lax.axis_index("s")
        base = c * HALF

        def zrow(r, carry):
            for k in range(D // 16):
                vals_v[r, pl.ds(k * 16, 16)] = jnp.zeros((16,), jnp.float32)
            return carry

        lax.fori_loop(0, 128, zrow, 0)
        for z in range(ZCH):
            pltpu.sync_copy(vals_v, acc_sh.at[pl.ds(s * TROWS + z * 128, 128)])
        plsc.subcore_barrier()

        def chunk(i, carry):
            g = s * SCH + i
            pltpu.sync_copy(idx_hbm.at[g], idx_v)
            pltpu.sync_copy(vals_hbm.at[pl.ds(g * 128, 128)], vals_v)
            for k in range(8):
                v = idx_v[pl.ds(k * 16, 16)]
                loc = v - base
                ok = (loc >= 0) & (loc < HALF)
                lidx_v[pl.ds(k * 16, 16)] = jnp.where(ok, loc, HALF)
            pltpu.sync_copy(vals_v, acc_sh.at[lidx_v], add=True)
            return carry

        lax.fori_loop(0, SCH, chunk, 0)
        plsc.subcore_barrier()
        for z in range(ZCH):
            off = s * TROWS + z * 128
            pltpu.sync_copy(acc_sh.at[pl.ds(off, 128)],
                            out_hbm.at[pl.ds(c * SROWS + off, 128)])

    return pl.kernel(
        body,
        out_type=jax.ShapeDtypeStruct((OUTR, D), jnp.float32),
        mesh=mesh,
        scratch_types=[
            pltpu.VMEM((128,), jnp.int32),
            pltpu.VMEM((128,), jnp.int32),
            pltpu.VMEM((128, D), jnp.float32),
            pltpu.VMEM_SHARED((SROWS, D), jnp.float32),
            pltpu.SemaphoreType.DMA,
        ],
        compiler_params=_SC_PARAMS,
    )(vals, idx2d)


def _mk_gather(D):
    @jax.custom_vjp
    def g(table, idx2d):
        return _sc_gather(table, idx2d, D)

    def fwd(table, idx2d):
        return _sc_gather(table, idx2d, D), idx2d

    def bwd(idx2d, ct):
        y = _sc_scatter(ct, idx2d, D)
        gt = jnp.concatenate([y[:HALF], y[SROWS:SROWS + HALF],
                              jnp.zeros((1, D), jnp.float32)], axis=0)
        return gt, _f0(idx2d)

    g.defvjp(fwd, bwd)
    return g


def _mk_scatter(D):
    @jax.custom_vjp
    def s(vals, idx2d):
        y = _sc_scatter(vals, idx2d, D)
        return jnp.concatenate([y[:HALF], y[SROWS:SROWS + HALF]], axis=0)

    def fwd(vals, idx2d):
        return s(vals, idx2d), idx2d

    def bwd(idx2d, ct):
        t = jnp.concatenate([ct, jnp.zeros((1, ct.shape[1]), jnp.float32)], 0)
        return _sc_gather(t, idx2d, ct.shape[1]), _f0(idx2d)

    s.defvjp(fwd, bwd)
    return s


_gath16 = _mk_gather(16)
_gath64 = _mk_gather(64)
_scat16 = _mk_scatter(16)
_scat64 = _mk_scatter(64)


def _gather(x, idx2d, g):
    table = jnp.concatenate([x, jnp.zeros((1, x.shape[1]), jnp.float32)], 0)
    return g(table, idx2d)


def _rowwise_call(f, arrs, consts, out_dims, br):
    R = arrs[0].shape[0]
    na, nc = len(arrs), len(consts)
    in_specs = [pl.BlockSpec((br, a.shape[1]), lambda i: (i, 0)) for a in arrs]
    in_specs += [pl.BlockSpec(c.shape, (lambda nd: (lambda i: (0,) * nd))(c.ndim))
                 for c in consts]
    out_specs = [pl.BlockSpec((br, od), lambda i: (i, 0)) for od in out_dims]
    out_shape = [jax.ShapeDtypeStruct((R, od), jnp.float32) for od in out_dims]

    def body(*refs):
        ivals = [r[...] for r in refs[:na + nc]]
        outs = f(*ivals)
        for oref, ov in zip(refs[na + nc:], outs):
            oref[...] = ov

    return pl.pallas_call(body, grid=(R // br,), in_specs=in_specs,
                          out_specs=out_specs, out_shape=out_shape)(
                              *arrs, *consts)


def _tc_op(f, arrs, consts, out_dims, br):
    arrs = tuple(arrs)
    consts = tuple(consts)
    na, nc, no = len(arrs), len(consts), len(out_dims)

    @jax.custom_vjp
    def op(*args):
        return tuple(_rowwise_call(f, args[:na], args[na:], out_dims, br))

    def fwd(*args):
        return op(*args), args

    def bwd(args, cts):
        aa, cc = args[:na], args[na:]

        def fb(*blocks):
            a = blocks[:na]
            g = blocks[na:na + no]
            cs = blocks[na + no:]
            _, vjpf = jax.vjp(lambda *xs: f(*xs, *cs), *a)
            return tuple(vjpf(tuple(g)))

        garr = _rowwise_call(fb, aa + tuple(cts), cc,
                             [a.shape[1] for a in aa], br)
        return tuple(garr) + tuple(jnp.zeros_like(c) for c in cc)

    op.defvjp(fwd, bwd)
    return op(*arrs, *consts)


def _swish(x):
    return x * jax.nn.sigmoid(x)


def _f_stage1(sx, sy, sz, srho, dx, dy, dz):
    ddx, ddy, ddz = sx - dx, sy - dy, sz - dz
    dist = jnp.sqrt(ddx * ddx + ddy * ddy + ddz * ddz + EPS)
    integ = 0.5 * (1.0 / jnp.maximum(dist, 0.05) - 1.0 / (dist + srho))
    cut = jnp.where(dist < RADIUS,
                    0.5 * (jnp.cos(jnp.pi * dist / RADIUS) + 1.0), 0.0)
    return dist, cut, integ


def _f_pre(x, top, bot):
    return (jnp.dot(x, top, preferred_element_type=jnp.float32),
            jnp.dot(x, bot, preferred_element_type=jnp.float32))


def _f_edge(gs, gd, cut, c1, wm2, b2):
    h1 = _swish(gs + gd + c1)
    h2 = _swish(jnp.dot(h1, wm2, preferred_element_type=jnp.float32) + b2)
    return (h2 * cut,)


def _f_node(act, agg, wu1, c2, wu2, bu2):
    u = _swish(jnp.dot(agg, wu1, preferred_element_type=jnp.float32) + c2)
    o = jnp.dot(u, wu2, preferred_element_type=jnp.float32) + bu2
    return (_swish(o) if act else o,)


def _f_epair(dist, bs, bd, qs, qd):
    d2 = dist * dist
    bij = bs * bd
    fgb = jnp.sqrt(d2 + bij * jnp.exp(-d2 / (4.0 * bij)))
    return (-0.5 * 138.935 * qs * qd / fgb,)


def _pln(a):
    return a.reshape(NCH, 128)


def _total_energy(pos, af, ei, si, p):
    pad = EP - E
    srcp = jnp.concatenate([ei[0], jnp.full((pad,), N, jnp.int32)])
    dstp = jnp.concatenate([ei[1], jnp.full((pad,), N, jnp.int32)])
    src2d = srcp.reshape(NCH, 128)
    dst2d = dstp.reshape(NCH, 128)

    charge = af[:, 0]
    rho = jnp.abs(af[:, 1]) * 0.05 + 0.12
    token = p['solv_emb'][si[0]]
    gamma = p['gamma_emb'][si[0], 0]

    geo = jnp.concatenate(
        [pos, charge[:, None], rho[:, None], jnp.zeros((N, 11), jnp.float32)],
        axis=1)
    gsrc = _gather(geo, src2d, _gath16)
    gdst = _gather(geo, dst2d, _gath16)

    dist_pl, cut_pl, integ_pl = _tc_op(
        _f_stage1,
        [_pln(gsrc[:, 0]), _pln(gsrc[:, 1]), _pln(gsrc[:, 2]),
         _pln(gsrc[:, 4]),
         _pln(gdst[:, 0]), _pln(gdst[:, 1]), _pln(gdst[:, 2])],
        [], [128, 128, 128], 64)

    integ16 = jnp.pad(integ_pl.reshape(EP, 1), ((0, 0), (0, 15)))
    I = _scat16(integ16, dst2d)[:, 0]
    Binv = 1.0 / rho - jnp.tanh(I * rho) / rho
    B = 1.0 / jnp.maximum(Binv, 1e-2)

    cut1 = cut_pl.reshape(EP, 1)
    x = jnp.pad(af, ((0, 0), (0, 5)))
    for li in range(3):
        wm1 = p['L%d_Wm1' % li]
        din = wm1.shape[0] // 2
        top, bot = wm1[:din], wm1[din:]
        if li == 0:
            top = jnp.pad(top, ((0, 5), (0, 0)))
            bot = jnp.pad(bot, ((0, 5), (0, 0)))
        As, Ad = _tc_op(_f_pre, [x], [top, bot], [HID, HID], 400)
        Gs = _gather(As, src2d, _gath64)
        Gd = _gather(Ad, dst2d, _gath64)
        c1 = (p['L%d_bm1' % li] + token @ p['L%d_Wt1' % li]).reshape(1, HID)
        b2 = p['L%d_bm2' % li].reshape(1, HID)
        (hcut,) = _tc_op(_f_edge, [Gs, Gd, cut1],
                         [c1, p['L%d_Wm2' % li], b2], [HID], 512)
        agg = _scat64(hcut, dst2d)
        c2 = (p['L%d_bu1' % li] + token @ p['L%d_Wt2' % li]).reshape(1, HID)
        wu2 = p['L%d_Wu2' % li]
        bu2 = p['L%d_bu2' % li]
        if li < 2:
            (x,) = _tc_op(functools.partial(_f_node, True), [agg],
                          [p['L%d_Wu1' % li], c2, wu2,
                           bu2.reshape(1, HID)], [HID], 400)
        else:
            wu2p = jnp.pad(wu2, ((0, 0), (0, 126)))
            bu2p = jnp.pad(bu2, (0, 126)).reshape(1, 128)
            (out2,) = _tc_op(functools.partial(_f_node, False), [agg],
                             [p['L%d_Wu1' % li], c2, wu2p, bu2p], [128], 400)

    scale = 1.0 + FRACTION * SCALING * (jax.nn.sigmoid(out2[:, 0]) - 0.5)
    Bn = B * scale
    bt = jnp.concatenate([Bn[:, None], jnp.zeros((N, 15), jnp.float32)], 1)
    gbs = _gather(bt, src2d, _gath16)
    gbd = _gather(bt, dst2d, _gath16)

    (ep_pl,) = _tc_op(
        _f_epair,
        [dist_pl, _pln(gbs[:, 0]), _pln(gbd[:, 0]),
         _pln(gsrc[:, 3]), _pln(gdst[:, 3])],
        [], [128], 64)
    ep16 = jnp.pad(ep_pl.reshape(EP, 1), ((0, 0), (0, 15)))
    e_node = _scat16(ep16, dst2d)[:, 0]

    e_self = -0.5 * 138.935 * charge * charge / Bn
    e_sa = gamma * (rho + 0.14) ** 2 * jax.nn.sigmoid(out2[:, 1])
    return jnp.sum(e_node + e_self + e_sa)


def kernel(pos, atom_features, edge_index, solvent_index, params):
    val, g = jax.value_and_grad(_total_energy)(
        pos, atom_features, edge_index, solvent_index, params)
    return val.reshape(1, 1), -g


# speedup vs baseline: 2.4820x; 1.0848x over previous
import functools

import jax
import jax.numpy as jnp
import numpy as np
from jax import lax
from jax.experimental import pallas as pl
from jax.experimental.pallas import tpu as pltpu
from jax.experimental.pallas import tpu_sc as plsc

N = 50000
E = 800000
HID = 64
RADIUS = 0.6
FRACTION = 0.1
SCALING = 2.0
EPS = 1e-6

EP = 802816
NCH = EP // 128
GCH = NCH // 32
SCH = NCH // 16
NT = N + 1
HALF = 25000
TROWS = 1664
ZCH = TROWS // 128
SROWS = 16 * TROWS
OUTR = 2 * SROWS

_MESH = dict(core_axis_name="c", subcore_axis_name="s", num_cores=2,
             num_subcores=16)
_SC_PARAMS = pltpu.CompilerParams(use_tc_tiling_on_sc=False)


def _f0(a):
    return np.zeros(a.shape, dtype=jax.dtypes.float0)


def _sc_gather(table, idx2d, D):
    mesh = plsc.VectorSubcoreMesh(**_MESH)
    NB = 4

    def body(table_hbm, idx_hbm, out_hbm, idxs_v, rows_v, gsem, wsem):
        wid = lax.axis_index("s") * 2 + lax.axis_index("c")
        pltpu.sync_copy(idx_hbm.at[pl.ds(wid * GCH, GCH)], idxs_v)

        def group(gi, carry):
            gds = []
            for b in range(NB):
                g = gi * NB + b
                gds.append(pltpu.async_copy(table_hbm.at[idxs_v.at[g]],
                                            rows_v.at[b], gsem))
            wds = []
            for b in range(NB):
                g = gi * NB + b
                gds[b].wait()
                row0 = (wid * GCH + g) * 128
                wds.append(pltpu.async_copy(rows_v.at[b],
                                            out_hbm.at[pl.ds(row0, 128)],
                                            wsem))
            for d in wds:
                d.wait()
            return carry

        lax.fori_loop(0, GCH // NB, group, 0)

    return pl.kernel(
        body,
        out_type=jax.ShapeDtypeStruct((EP, D), jnp.float32),
        mesh=mesh,
        scratch_types=[
            pltpu.VMEM((GCH, 128), jnp.int32),
            pltpu.VMEM((NB, 128, D), jnp.float32),
            pltpu.SemaphoreType.DMA,
            pltpu.SemaphoreType.DMA,
        ],
        compiler_params=_SC_PARAMS,
    )(table, idx2d)


def _sc_scatter(vals, idx2d, D):
    mesh = plsc.VectorSubcoreMesh(**_MESH)

    NB = 2
    SEG = 56
    NSEG = SCH // SEG

    def body(vals_hbm, idx_hbm, out_hbm, idxs_v, lidx_v, vals_v, acc_sh,
             vsem, ssem):
        c = lax.axis_index("c")
        s = lax.axis_index("s")
        base = c * HALF

        def zrow(r, carry):
            for k in range(D // 16):
                vals_v[0, r, pl.ds(k * 16, 16)] = jnp.zeros((16,), jnp.float32)
            return carry

        lax.fori_loop(0, 128, zrow, 0)
        for z in range(ZCH):
            pltpu.sync_copy(vals_v.at[0],
                            acc_sh.at[pl.ds(s * TROWS + z * 128, 128)])
        plsc.subcore_barrier()

        def seg(si, carry):
            pltpu.sync_copy(idx_hbm.at[pl.ds(s * SCH + si * SEG, SEG)],
                            idxs_v)

            def group(gi, carry2):
                vds = []
                for b in range(NB):
                    j = gi * NB + b
                    row0 = (s * SCH + si * SEG + j) * 128
                    vds.append(pltpu.async_copy(
                        vals_hbm.at[pl.ds(row0, 128)], vals_v.at[b], vsem))
                sds = []
                for b in range(NB):
                    j = gi * NB + b
                    vds[b].wait()
                    for k in range(8):
                        v = idxs_v[j, pl.ds(k * 16, 16)]
                        loc = v - base
                        ok = (loc >= 0) & (loc < HALF)
                        lidx_v[b, pl.ds(k * 16, 16)] = jnp.where(ok, loc, HALF)
                    sds.append(pltpu.async_copy(vals_v.at[b],
                                                acc_sh.at[lidx_v.at[b]],
                                                ssem, add=True))
                for d in sds:
                    d.wait()
                return carry2

            lax.fori_loop(0, SEG // NB, group, 0)
            return carry

        lax.fori_loop(0, NSEG, seg, 0)
        plsc.subcore_barrier()
        for z in range(ZCH):
            off = s * TROWS + z * 128
            pltpu.sync_copy(acc_sh.at[pl.ds(off, 128)],
                            out_hbm.at[pl.ds(c * SROWS + off, 128)])

    return pl.kernel(
        body,
        out_type=jax.ShapeDtypeStruct((OUTR, D), jnp.float32),
        mesh=mesh,
        scratch_types=[
            pltpu.VMEM((SEG, 128), jnp.int32),
            pltpu.VMEM((NB, 128), jnp.int32),
            pltpu.VMEM((NB, 128, D), jnp.float32),
            pltpu.VMEM_SHARED((SROWS, D), jnp.float32),
            pltpu.SemaphoreType.DMA,
            pltpu.SemaphoreType.DMA,
        ],
        compiler_params=_SC_PARAMS,
    )(vals, idx2d)


def _mk_gather(D):
    @jax.custom_vjp
    def g(table, idx2d):
        return _sc_gather(table, idx2d, D)

    def fwd(table, idx2d):
        return _sc_gather(table, idx2d, D), idx2d

    def bwd(idx2d, ct):
        y = _sc_scatter(ct, idx2d, D)
        gt = jnp.concatenate([y[:HALF], y[SROWS:SROWS + HALF],
                              jnp.zeros((1, D), jnp.float32)], axis=0)
        return gt, _f0(idx2d)

    g.defvjp(fwd, bwd)
    return g


def _mk_scatter(D):
    @jax.custom_vjp
    def s(vals, idx2d):
        y = _sc_scatter(vals, idx2d, D)
        return jnp.concatenate([y[:HALF], y[SROWS:SROWS + HALF]], axis=0)

    def fwd(vals, idx2d):
        return s(vals, idx2d), idx2d

    def bwd(idx2d, ct):
        t = jnp.concatenate([ct, jnp.zeros((1, ct.shape[1]), jnp.float32)], 0)
        return _sc_gather(t, idx2d, ct.shape[1]), _f0(idx2d)

    s.defvjp(fwd, bwd)
    return s


_gath16 = _mk_gather(16)
_gath64 = _mk_gather(64)
_scat16 = _mk_scatter(16)
_scat64 = _mk_scatter(64)


def _gather(x, idx2d, g):
    table = jnp.concatenate([x, jnp.zeros((1, x.shape[1]), jnp.float32)], 0)
    return g(table, idx2d)


def _rowwise_call(f, arrs, consts, out_dims, br):
    R = arrs[0].shape[0]
    na, nc = len(arrs), len(consts)
    in_specs = [pl.BlockSpec((br, a.shape[1]), lambda i: (i, 0)) for a in arrs]
    in_specs += [pl.BlockSpec(c.shape, (lambda nd: (lambda i: (0,) * nd))(c.ndim))
                 for c in consts]
    out_specs = [pl.BlockSpec((br, od), lambda i: (i, 0)) for od in out_dims]
    out_shape = [jax.ShapeDtypeStruct((R, od), jnp.float32) for od in out_dims]

    def body(*refs):
        ivals = [r[...] for r in refs[:na + nc]]
        outs = f(*ivals)
        for oref, ov in zip(refs[na + nc:], outs):
            oref[...] = ov

    return pl.pallas_call(body, grid=(R // br,), in_specs=in_specs,
                          out_specs=out_specs, out_shape=out_shape)(
                              *arrs, *consts)


def _tc_op(f, arrs, consts, out_dims, br):
    arrs = tuple(arrs)
    consts = tuple(consts)
    na, nc, no = len(arrs), len(consts), len(out_dims)

    @jax.custom_vjp
    def op(*args):
        return tuple(_rowwise_call(f, args[:na], args[na:], out_dims, br))

    def fwd(*args):
        return op(*args), args

    def bwd(args, cts):
        aa, cc = args[:na], args[na:]

        def fb(*blocks):
            a = blocks[:na]
            g = blocks[na:na + no]
            cs = blocks[na + no:]
            _, vjpf = jax.vjp(lambda *xs: f(*xs, *cs), *a)
            return tuple(vjpf(tuple(g)))

        garr = _rowwise_call(fb, aa + tuple(cts), cc,
                             [a.shape[1] for a in aa], br)
        return tuple(garr) + tuple(jnp.zeros_like(c) for c in cc)

    op.defvjp(fwd, bwd)
    return op(*arrs, *consts)


def _swish(x):
    return x * jax.nn.sigmoid(x)


def _f_stage1(sx, sy, sz, srho, dx, dy, dz):
    ddx, ddy, ddz = sx - dx, sy - dy, sz - dz
    dist = jnp.sqrt(ddx * ddx + ddy * ddy + ddz * ddz + EPS)
    integ = 0.5 * (1.0 / jnp.maximum(dist, 0.05) - 1.0 / (dist + srho))
    cut = jnp.where(dist < RADIUS,
                    0.5 * (jnp.cos(jnp.pi * dist / RADIUS) + 1.0), 0.0)
    return dist, cut, integ


def _f_pre(x, top, bot):
    return (jnp.dot(x, top, preferred_element_type=jnp.float32),
            jnp.dot(x, bot, preferred_element_type=jnp.float32))


def _f_edge(gs, gd, cut, c1, wm2, b2):
    h1 = _swish(gs + gd + c1)
    h2 = _swish(jnp.dot(h1, wm2, preferred_element_type=jnp.float32) + b2)
    return (h2 * cut,)


def _f_node(act, agg, wu1, c2, wu2, bu2):
    u = _swish(jnp.dot(agg, wu1, preferred_element_type=jnp.float32) + c2)
    o = jnp.dot(u, wu2, preferred_element_type=jnp.float32) + bu2
    return (_swish(o) if act else o,)


def _f_epair(dist, bs, bd, qs, qd):
    d2 = dist * dist
    bij = bs * bd
    fgb = jnp.sqrt(d2 + bij * jnp.exp(-d2 / (4.0 * bij)))
    return (-0.5 * 138.935 * qs * qd / fgb,)


def _pln(a):
    return a.reshape(NCH, 128)


def _total_energy(pos, af, ei, si, p):
    pad = EP - E
    srcp = jnp.concatenate([ei[0], jnp.full((pad,), N, jnp.int32)])
    dstp = jnp.concatenate([ei[1], jnp.full((pad,), N, jnp.int32)])
    src2d = srcp.reshape(NCH, 128)
    dst2d = dstp.reshape(NCH, 128)

    charge = af[:, 0]
    rho = jnp.abs(af[:, 1]) * 0.05 + 0.12
    token = p['solv_emb'][si[0]]
    gamma = p['gamma_emb'][si[0], 0]

    geo = jnp.concatenate(
        [pos, charge[:, None], rho[:, None], jnp.zeros((N, 11), jnp.float32)],
        axis=1)
    gsrc = _gather(geo, src2d, _gath16)
    gdst = _gather(geo, dst2d, _gath16)

    dist_pl, cut_pl, integ_pl = _tc_op(
        _f_stage1,
        [_pln(gsrc[:, 0]), _pln(gsrc[:, 1]), _pln(gsrc[:, 2]),
         _pln(gsrc[:, 4]),
         _pln(gdst[:, 0]), _pln(gdst[:, 1]), _pln(gdst[:, 2])],
        [], [128, 128, 128], 64)

    integ16 = jnp.pad(integ_pl.reshape(EP, 1), ((0, 0), (0, 15)))
    I = _scat16(integ16, dst2d)[:, 0]
    Binv = 1.0 / rho - jnp.tanh(I * rho) / rho
    B = 1.0 / jnp.maximum(Binv, 1e-2)

    cut1 = cut_pl.reshape(EP, 1)
    x = jnp.pad(af, ((0, 0), (0, 5)))
    for li in range(3):
        wm1 = p['L%d_Wm1' % li]
        din = wm1.shape[0] // 2
        top, bot = wm1[:din], wm1[din:]
        if li == 0:
            top = jnp.pad(top, ((0, 5), (0, 0)))
            bot = jnp.pad(bot, ((0, 5), (0, 0)))
        As, Ad = _tc_op(_f_pre, [x], [top, bot], [HID, HID], 400)
        Gs = _gather(As, src2d, _gath64)
        Gd = _gather(Ad, dst2d, _gath64)
        c1 = (p['L%d_bm1' % li] + token @ p['L%d_Wt1' % li]).reshape(1, HID)
        b2 = p['L%d_bm2' % li].reshape(1, HID)
        (hcut,) = _tc_op(_f_edge, [Gs, Gd, cut1],
                         [c1, p['L%d_Wm2' % li], b2], [HID], 512)
        agg = _scat64(hcut, dst2d)
        c2 = (p['L%d_bu1' % li] + token @ p['L%d_Wt2' % li]).reshape(1, HID)
        wu2 = p['L%d_Wu2' % li]
        bu2 = p['L%d_bu2' % li]
        if li < 2:
            (x,) = _tc_op(functools.partial(_f_node, True), [agg],
                          [p['L%d_Wu1' % li], c2, wu2,
                           bu2.reshape(1, HID)], [HID], 400)
        else:
            wu2p = jnp.pad(wu2, ((0, 0), (0, 126)))
            bu2p = jnp.pad(bu2, (0, 126)).reshape(1, 128)
            (out2,) = _tc_op(functools.partial(_f_node, False), [agg],
                             [p['L%d_Wu1' % li], c2, wu2p, bu2p], [128], 400)

    scale = 1.0 + FRACTION * SCALING * (jax.nn.sigmoid(out2[:, 0]) - 0.5)
    Bn = B * scale
    bt = jnp.concatenate([Bn[:, None], jnp.zeros((N, 15), jnp.float32)], 1)
    gbs = _gather(bt, src2d, _gath16)
    gbd = _gather(bt, dst2d, _gath16)

    (ep_pl,) = _tc_op(
        _f_epair,
        [dist_pl, _pln(gbs[:, 0]), _pln(gbd[:, 0]),
         _pln(gsrc[:, 3]), _pln(gdst[:, 3])],
        [], [128], 64)
    ep16 = jnp.pad(ep_pl.reshape(EP, 1), ((0, 0), (0, 15)))
    e_node = _scat16(ep16, dst2d)[:, 0]

    e_self = -0.5 * 138.935 * charge * charge / Bn
    e_sa = gamma * (rho + 0.14) ** 2 * jax.nn.sigmoid(out2[:, 1])
    return jnp.sum(e_node + e_self + e_sa)


def kernel(pos, atom_features, edge_index, solvent_index, params):
    val, g = jax.value_and_grad(_total_energy)(
        pos, atom_features, edge_index, solvent_index, params)
    return val.reshape(1, 1), -g
